# Initial kernel scaffold; baseline (speedup 1.0000x reference)
import jax
import jax.numpy as jnp
from jax.experimental import pallas as pl


def kernel(x, edge_index, W1, b1, W2, b2, Wl, bl):
    raise NotImplementedError("write your pallas kernel here")


# speedup vs baseline: 24.1146x; 24.1146x over previous
import functools

import jax
import jax.numpy as jnp
from jax import lax
from jax.experimental import pallas as pl
from jax.experimental.pallas import tpu as pltpu
from jax.experimental.pallas import tpu_sc as plsc

N = 10000
E = 320000
D = 128
NC = 2
NS = 16
NW = NC * NS
EPW = E // NW
CH = 80
NCHUNK = EPW // CH
STRIPE = 640
TAIL = N - (NS - 1) * STRIPE

_F32 = jnp.float32


def _sc_mesh():
    return plsc.VectorSubcoreMesh(core_axis_name="c", subcore_axis_name="s")


@functools.partial(
    pl.kernel,
    out_type=jax.ShapeDtypeStruct((NC, N), _F32),
    mesh=_sc_mesh(),
    scratch_types=[
        pltpu.VMEM((NCHUNK, CH), jnp.int32),
        pltpu.VMEM((CH,), _F32),
        pltpu.VMEM_SHARED((N,), _F32),
    ],
)
def _deg_sc(dst_hbm, z1_hbm, out_hbm, idx_v, ones_v, acc):
    c = lax.axis_index("c")
    s = lax.axis_index("s")
    wid = c * NS + s
    pltpu.sync_copy(dst_hbm.at[wid], idx_v)
    for i in range(CH // 16):
        ones_v[pl.ds(i * 16, 16)] = jnp.ones((16,), _F32)

    @pl.when(s == 0)
    def _zero():
        pltpu.sync_copy(z1_hbm, acc)

    plsc.subcore_barrier()

    def body(j, carry):
        pltpu.sync_copy(ones_v, acc.at[idx_v.at[j]], add=True)
        return carry

    lax.fori_loop(0, NCHUNK, body, 0)
    plsc.subcore_barrier()

    @pl.when(s == 0)
    def _out():
        pltpu.sync_copy(acc, out_hbm.at[c])


@functools.partial(
    pl.kernel,
    out_type=jax.ShapeDtypeStruct((NC, N, D), _F32),
    mesh=_sc_mesh(),
    scratch_types=[
        pltpu.VMEM((NCHUNK, CH), jnp.int32),
        pltpu.VMEM((NCHUNK, CH), jnp.int32),
        pltpu.VMEM((CH, D), _F32),
        pltpu.SemaphoreType.DMA,
        pltpu.VMEM_SHARED((N, D), _F32),
    ],
)
def _agg_sc(y_hbm, src_hbm, dst_hbm, z2_hbm, out_hbm, src_v, dst_v, buf, sem, acc):
    c = lax.axis_index("c")
    s = lax.axis_index("s")
    wid = c * NS + s
    pltpu.sync_copy(src_hbm.at[wid], src_v)
    pltpu.sync_copy(dst_hbm.at[wid], dst_v)

    @pl.when(s < NS - 1)
    def _zero_main():
        pltpu.sync_copy(z2_hbm, acc.at[pl.ds(s * STRIPE, STRIPE)])

    @pl.when(s == NS - 1)
    def _zero_tail():
        pltpu.sync_copy(z2_hbm.at[pl.ds(0, TAIL)],
                        acc.at[pl.ds((NS - 1) * STRIPE, TAIL)])

    plsc.subcore_barrier()

    def body(j, carry):
        pltpu.async_copy(y_hbm.at[src_v.at[j]], buf, sem).wait()
        pltpu.sync_copy(buf, acc.at[dst_v.at[j]], add=True)
        return carry

    lax.fori_loop(0, NCHUNK, body, 0)
    plsc.subcore_barrier()

    @pl.when(s < NS - 1)
    def _out_main():
        pltpu.sync_copy(acc.at[pl.ds(s * STRIPE, STRIPE)],
                        out_hbm.at[c, pl.ds(s * STRIPE, STRIPE)])

    @pl.when(s == NS - 1)
    def _out_tail():
        pltpu.sync_copy(acc.at[pl.ds((NS - 1) * STRIPE, TAIL)],
                        out_hbm.at[c, pl.ds((NS - 1) * STRIPE, TAIL)])


@functools.partial(
    pl.kernel,
    out_type=jax.ShapeDtypeStruct((NC, N), _F32),
    mesh=_sc_mesh(),
    scratch_types=[
        pltpu.VMEM((NCHUNK, CH), jnp.int32),
        pltpu.VMEM((NCHUNK, CH), jnp.int32),
        pltpu.VMEM((CH,), _F32),
        pltpu.SemaphoreType.DMA,
        pltpu.VMEM_SHARED((N,), _F32),
    ],
)
def _sagg_sc(zp_hbm, src_hbm, dst_hbm, z1_hbm, out_hbm, src_v, dst_v, buf, sem, acc):
    c = lax.axis_index("c")
    s = lax.axis_index("s")
    wid = c * NS + s
    pltpu.sync_copy(src_hbm.at[wid], src_v)
    pltpu.sync_copy(dst_hbm.at[wid], dst_v)

    @pl.when(s == 0)
    def _zero():
        pltpu.sync_copy(z1_hbm, acc)

    plsc.subcore_barrier()

    def body(j, carry):
        pltpu.async_copy(zp_hbm.at[src_v.at[j]], buf, sem).wait()
        pltpu.sync_copy(buf, acc.at[dst_v.at[j]], add=True)
        return carry

    lax.fori_loop(0, NCHUNK, body, 0)
    plsc.subcore_barrier()

    @pl.when(s == 0)
    def _out():
        pltpu.sync_copy(acc, out_hbm.at[c])


def _mm_body(x_ref, w_ref, o_ref):
    o_ref[...] = jnp.dot(x_ref[...], w_ref[...], preferred_element_type=_F32)


_mm_tc = pl.pallas_call(_mm_body, out_shape=jax.ShapeDtypeStruct((N, D), _F32))


def _scale_body(deg_ref, xw_ref, dinv_ref, y_ref):
    deg = deg_ref[0] + deg_ref[1]
    dinv = 1.0 / jnp.sqrt(deg)
    dinv_ref[...] = dinv
    y_ref[...] = xw_ref[...] * dinv


_scale_tc = pl.pallas_call(
    _scale_body,
    out_shape=(
        jax.ShapeDtypeStruct((N, 1), _F32),
        jax.ShapeDtypeStruct((N, D), _F32),
    ),
)


def _mid_body(agg_ref, y_ref, dinv_ref, b1_ref, w2_ref, wl_ref, zp_ref):
    aggs = agg_ref[0] + agg_ref[1] + y_ref[...]
    h = jnp.maximum(aggs * dinv_ref[...] + b1_ref[...][None, :], 0.0)
    wv = jnp.dot(w2_ref[...], wl_ref[...], preferred_element_type=_F32)
    zp_ref[...] = jnp.dot(h, wv, preferred_element_type=_F32) * dinv_ref[...]


_mid_tc = pl.pallas_call(
    _mid_body,
    out_shape=jax.ShapeDtypeStruct((N, 1), _F32),
)


def _tail_body(sagg_ref, zp_ref, dinv_ref, b2_ref, wl_ref, bl_ref, o_ref):
    t = (sagg_ref[0] + sagg_ref[1] + zp_ref[...]) * dinv_ref[...]
    c0 = jnp.dot(b2_ref[...][None, :], wl_ref[...], preferred_element_type=_F32)
    t = t + c0 + bl_ref[...][None, :]
    sig = 1.0 / (1.0 + jnp.exp(-t))
    o_ref[...] = jnp.mean(sig, axis=0)


_tail_tc = pl.pallas_call(
    _tail_body,
    out_shape=jax.ShapeDtypeStruct((1,), _F32),
)


def kernel(x, edge_index, W1, b1, W2, b2, Wl, bl):
    src = edge_index[0].reshape(NW, NCHUNK, CH)
    dst = edge_index[1].reshape(NW, NCHUNK, CH)
    z1 = jnp.zeros((N,), _F32)
    z2 = jnp.zeros((STRIPE, D), _F32)

    deg = _deg_sc(dst, z1)
    xw = _mm_tc(x, W1)
    deg3 = deg.reshape(NC, N, 1)
    dinv, y = _scale_tc(deg3, xw)
    agg = _agg_sc(y, src, dst, z2)
    zp = _mid_tc(agg, y, dinv, b1, W2, Wl)
    sagg = _sagg_sc(zp.reshape(N), src, dst, z1)
    return _tail_tc(sagg.reshape(NC, N, 1), zp, dinv, b2, Wl, bl)


# speedup vs baseline: 40.7332x; 1.6892x over previous
import functools

import jax
import jax.numpy as jnp
from jax import lax
from jax.experimental import pallas as pl
from jax.experimental.pallas import tpu as pltpu
from jax.experimental.pallas import tpu_sc as plsc

N = 10000
E = 320000
D = 128
NC = 2
NS = 16
NW = NC * NS
EPW = E // NW
CH = 80
NCHUNK = EPW // CH
STRIPE = 640
TAIL = N - (NS - 1) * STRIPE

_F32 = jnp.float32


def _sc_mesh():
    return plsc.VectorSubcoreMesh(core_axis_name="c", subcore_axis_name="s")


def _edge_pipeline(eidx_hbm, wid, table, acc, iring, bufs, isems, gsems,
                   ni, ng):
    def issue_idx(j, islot):
        pltpu.async_copy(eidx_hbm.at[wid, j], iring.at[islot],
                         isems.at[islot])

    def wait_idx(j, islot):
        pltpu.make_async_copy(eidx_hbm.at[wid, j], iring.at[islot],
                              isems.at[islot]).wait()

    def issue_g(islot, gslot):
        pltpu.async_copy(table.at[iring.at[islot, 0]], bufs.at[gslot],
                         gsems.at[gslot])

    def wait_g(islot, gslot):
        pltpu.make_async_copy(table.at[iring.at[islot, 0]], bufs.at[gslot],
                              gsems.at[gslot]).wait()

    def scatter(islot, gslot):
        pltpu.sync_copy(bufs.at[gslot], acc.at[iring.at[islot, 1]], add=True)

    for j in range(ni):
        issue_idx(j, j)
    for j in range(ng):
        wait_idx(j, j)
        issue_g(j, j)

    n_groups = (NCHUNK - ni - ni + 1) // ni

    def body(g, carry):
        j0 = g * ni
        for b in range(ni):
            j = j0 + b
            wait_g(b % ni, b % ng)
            scatter(b % ni, b % ng)
            issue_idx(j + ni, b % ni)
            wait_idx(j + ng, (b + ng) % ni)
            issue_g((b + ng) % ni, b % ng)
        return carry

    lax.fori_loop(0, n_groups, body, 0)
    for j in range(n_groups * ni, NCHUNK):
        wait_g(j % ni, j % ng)
        scatter(j % ni, j % ng)
        if j + ni < NCHUNK:
            issue_idx(j + ni, j % ni)
        if j + ng < NCHUNK:
            wait_idx(j + ng, (j + ng) % ni)
            issue_g((j + ng) % ni, j % ng)


@functools.partial(
    pl.kernel,
    out_type=jax.ShapeDtypeStruct((NC, N), _F32),
    mesh=_sc_mesh(),
    scratch_types=[
        pltpu.VMEM((NCHUNK, 2, CH), jnp.int32),
        pltpu.VMEM((CH,), _F32),
        pltpu.VMEM_SHARED((N,), _F32),
    ],
)
def _deg_sc(eidx_hbm, z1_hbm, out_hbm, idx_v, ones_v, acc):
    c = lax.axis_index("c")
    s = lax.axis_index("s")
    wid = c * NS + s
    pltpu.sync_copy(eidx_hbm.at[wid], idx_v)
    for i in range(CH // 16):
        ones_v[pl.ds(i * 16, 16)] = jnp.ones((16,), _F32)

    @pl.when(s == 0)
    def _zero():
        pltpu.sync_copy(z1_hbm, acc)

    plsc.subcore_barrier()

    def body(j, carry):
        pltpu.sync_copy(ones_v, acc.at[idx_v.at[j, 1]], add=True)
        return carry

    lax.fori_loop(0, NCHUNK, body, 0)
    plsc.subcore_barrier()

    @pl.when(s == 0)
    def _out():
        pltpu.sync_copy(acc, out_hbm.at[c])


_AGG_NI = 6
_AGG_NG = 3


@functools.partial(
    pl.kernel,
    out_type=jax.ShapeDtypeStruct((NC, N, D), _F32),
    mesh=_sc_mesh(),
    scratch_types=[
        pltpu.VMEM((_AGG_NI, 2, CH), jnp.int32),
        pltpu.VMEM((_AGG_NG, CH, D), _F32),
        pltpu.SemaphoreType.DMA((_AGG_NI,)),
        pltpu.SemaphoreType.DMA((_AGG_NG,)),
        pltpu.VMEM_SHARED((N, D), _F32),
    ],
)
def _agg_sc(y_hbm, eidx_hbm, z2_hbm, out_hbm, iring, bufs, isems, gsems, acc):
    c = lax.axis_index("c")
    s = lax.axis_index("s")
    wid = c * NS + s

    @pl.when(s < NS - 1)
    def _zero_main():
        pltpu.sync_copy(z2_hbm, acc.at[pl.ds(s * STRIPE, STRIPE)])

    @pl.when(s == NS - 1)
    def _zero_tail():
        pltpu.sync_copy(z2_hbm.at[pl.ds(0, TAIL)],
                        acc.at[pl.ds((NS - 1) * STRIPE, TAIL)])

    plsc.subcore_barrier()
    _edge_pipeline(eidx_hbm, wid, y_hbm, acc, iring, bufs, isems, gsems,
                   _AGG_NI, _AGG_NG)
    plsc.subcore_barrier()

    @pl.when(s < NS - 1)
    def _out_main():
        pltpu.sync_copy(acc.at[pl.ds(s * STRIPE, STRIPE)],
                        out_hbm.at[c, pl.ds(s * STRIPE, STRIPE)])

    @pl.when(s == NS - 1)
    def _out_tail():
        pltpu.sync_copy(acc.at[pl.ds((NS - 1) * STRIPE, TAIL)],
                        out_hbm.at[c, pl.ds((NS - 1) * STRIPE, TAIL)])


_SAG_NI = 10
_SAG_NG = 5


@functools.partial(
    pl.kernel,
    out_type=jax.ShapeDtypeStruct((NC, N), _F32),
    mesh=_sc_mesh(),
    scratch_types=[
        pltpu.VMEM((_SAG_NI, 2, CH), jnp.int32),
        pltpu.VMEM((_SAG_NG, CH), _F32),
        pltpu.SemaphoreType.DMA((_SAG_NI,)),
        pltpu.SemaphoreType.DMA((_SAG_NG,)),
        pltpu.VMEM_SHARED((N,), _F32),
    ],
)
def _sagg_sc(zp_hbm, eidx_hbm, z1_hbm, out_hbm, iring, bufs, isems, gsems,
             acc):
    c = lax.axis_index("c")
    s = lax.axis_index("s")
    wid = c * NS + s

    @pl.when(s == 0)
    def _zero():
        pltpu.sync_copy(z1_hbm, acc)

    plsc.subcore_barrier()
    _edge_pipeline(eidx_hbm, wid, zp_hbm, acc, iring, bufs, isems, gsems,
                   _SAG_NI, _SAG_NG)
    plsc.subcore_barrier()

    @pl.when(s == 0)
    def _out():
        pltpu.sync_copy(acc, out_hbm.at[c])


def _mm_body(x_ref, w_ref, o_ref):
    o_ref[...] = jnp.dot(x_ref[...], w_ref[...], preferred_element_type=_F32)


_mm_tc = pl.pallas_call(_mm_body, out_shape=jax.ShapeDtypeStruct((N, D), _F32))


def _scale_body(deg_ref, xw_ref, dinv_ref, y_ref):
    deg = deg_ref[0] + deg_ref[1]
    dinv = 1.0 / jnp.sqrt(deg)
    dinv_ref[...] = dinv
    y_ref[...] = xw_ref[...] * dinv


_scale_tc = pl.pallas_call(
    _scale_body,
    out_shape=(
        jax.ShapeDtypeStruct((N, 1), _F32),
        jax.ShapeDtypeStruct((N, D), _F32),
    ),
)


def _mid_body(agg_ref, y_ref, dinv_ref, b1_ref, w2_ref, wl_ref, zp_ref):
    aggs = agg_ref[0] + agg_ref[1] + y_ref[...]
    h = jnp.maximum(aggs * dinv_ref[...] + b1_ref[...][None, :], 0.0)
    wv = jnp.dot(w2_ref[...], wl_ref[...], preferred_element_type=_F32)
    zp_ref[...] = jnp.dot(h, wv, preferred_element_type=_F32) * dinv_ref[...]


_mid_tc = pl.pallas_call(
    _mid_body,
    out_shape=jax.ShapeDtypeStruct((N, 1), _F32),
)


def _tail_body(sagg_ref, zp_ref, dinv_ref, b2_ref, wl_ref, bl_ref, o_ref):
    t = (sagg_ref[0] + sagg_ref[1] + zp_ref[...]) * dinv_ref[...]
    c0 = jnp.dot(b2_ref[...][None, :], wl_ref[...], preferred_element_type=_F32)
    t = t + c0 + bl_ref[...][None, :]
    sig = 1.0 / (1.0 + jnp.exp(-t))
    o_ref[...] = jnp.mean(sig, axis=0)


_tail_tc = pl.pallas_call(
    _tail_body,
    out_shape=jax.ShapeDtypeStruct((1,), _F32),
)


def kernel(x, edge_index, W1, b1, W2, b2, Wl, bl):
    src = edge_index[0].reshape(NW, NCHUNK, CH)
    dst = edge_index[1].reshape(NW, NCHUNK, CH)
    eidx = jnp.stack([src, dst], axis=2)
    z1 = jnp.zeros((N,), _F32)
    z2 = jnp.zeros((STRIPE, D), _F32)

    deg = _deg_sc(eidx, z1)
    xw = _mm_tc(x, W1)
    deg3 = deg.reshape(NC, N, 1)
    dinv, y = _scale_tc(deg3, xw)
    agg = _agg_sc(y, eidx, z2)
    zp = _mid_tc(agg, y, dinv, b1, W2, Wl)
    sagg = _sagg_sc(zp.reshape(N), eidx, z1)
    return _tail_tc(sagg.reshape(NC, N, 1), zp, dinv, b2, Wl, bl)
